# Initial kernel scaffold; baseline (speedup 1.0000x reference)
import jax
import jax.numpy as jnp
from jax.experimental import pallas as pl


def kernel(atom_features, atom_pos, molecule_edges, vertex2molecule, num_molecules, W1_0, b1_0, W2_0, b2_0, bias_0, W1_1, b1_1, W2_1, b2_1, bias_1):
    raise NotImplementedError("write your pallas kernel here")


# speedup vs baseline: 2.6605x; 2.6605x over previous
import functools

import jax
import jax.numpy as jnp
from jax import lax
from jax.experimental import pallas as pl
from jax.experimental.pallas import tpu as pltpu
from jax.experimental.pallas import tpu_sc as plsc

RADIUS = 1.54
HID = 32
NUM_MOL = 500

NC, NS = 2, 16
NW = NC * NS
CH = 1024
IR = CH // 128

_sc_mesh = functools.partial(
    plsc.VectorSubcoreMesh, core_axis_name="c", subcore_axis_name="s",
    num_cores=NC, num_subcores=NS)
_sc_params = pltpu.CompilerParams(use_tc_tiling_on_sc=False)


def _worker_id():
    return lax.axis_index("s") * NC + lax.axis_index("c")


def _gather_edges_body(T, tsrc, posd, srcp, dstp, a_out, pd_out,
                       idx_s, idx_d, abuf, pbuf, sem, sem2):
    w = _worker_id()

    def chunk(t, carry):
        row0 = (w * T + t) * IR
        e0 = (w * T + t) * CH
        pltpu.sync_copy(srcp.at[pl.ds(row0, IR)], idx_s)
        pltpu.sync_copy(dstp.at[pl.ds(row0, IR)], idx_d)
        cps = [pltpu.async_copy(tsrc.at[idx_s.at[j]],
                                abuf.at[pl.ds(j * 128, 128)], sem)
               for j in range(IR)]
        cps += [pltpu.async_copy(posd.at[idx_d.at[j]],
                                 pbuf.at[pl.ds(j * 128, 128)], sem2)
                for j in range(IR)]
        for cp in cps:
            cp.wait()
        pltpu.sync_copy(abuf, a_out.at[pl.ds(e0, CH)])
        pltpu.sync_copy(pbuf, pd_out.at[pl.ds(e0, CH)])
        return carry

    lax.fori_loop(0, T, chunk, 0)


def _gather_table_body(T, table, srcp, out, idx_s, buf, sem):
    w = _worker_id()

    def chunk(t, carry):
        row0 = (w * T + t) * IR
        e0 = (w * T + t) * CH
        pltpu.sync_copy(srcp.at[pl.ds(row0, IR)], idx_s)
        cps = [pltpu.async_copy(table.at[idx_s.at[j]],
                                buf.at[pl.ds(j * 128, 128)], sem)
               for j in range(IR)]
        for cp in cps:
            cp.wait()
        pltpu.sync_copy(buf, out.at[pl.ds(e0, CH)])
        return carry

    lax.fori_loop(0, T, chunk, 0)


def _scatter_add_body(T, NP, msg, dstp, zeros, out, idx_d, mbuf, table, sem):
    c = lax.axis_index("c")
    s = lax.axis_index("s")
    w = s * NC + c
    rpt = NP // NS
    pltpu.sync_copy(zeros.at[pl.ds(s * rpt, rpt)],
                    table.at[pl.ds(s * rpt, rpt)])
    plsc.subcore_barrier()

    def chunk(t, carry):
        row0 = (w * T + t) * IR
        e0 = (w * T + t) * CH
        pltpu.sync_copy(dstp.at[pl.ds(row0, IR)], idx_d)
        pltpu.sync_copy(msg.at[pl.ds(e0, CH)], mbuf)
        for j in range(IR):
            pltpu.sync_copy(mbuf.at[pl.ds(j * 128, 128)],
                            table.at[idx_d.at[j]], add=True)
        return carry

    lax.fori_loop(0, T, chunk, 0)
    plsc.subcore_barrier()
    pltpu.sync_copy(table.at[pl.ds(s * rpt, rpt)],
                    out.at[c].at[pl.ds(s * rpt, rpt)])


def _edge0_body(a_ref, pd_ref, w10, b10, w2p, b0m, msg_ref, hood_ref):
    a = a_ref[...]
    pd = pd_ref[...]
    inv_r = 1.0 / RADIUS
    hood = (a[:, 7:9] - pd[:, 0:2]) * inv_r
    h0 = jnp.tanh(hood[:, 0:1] * w10[0:1, :] + hood[:, 1:2] * w10[1:2, :]
                  + b10[...])
    p = jnp.dot(h0, w2p[...], preferred_element_type=jnp.float32)
    f = a[:, 0:7]
    acc = jnp.dot(f, b0m[...], preferred_element_type=jnp.float32)
    for i in range(7):
        acc = acc + p[:, i * HID:(i + 1) * HID] * f[:, i:i + 1]
    msg_ref[...] = acc
    hood_ref[...] = hood


def _node_tanh_body(p_ref, b_ref, f_ref):
    p = p_ref[...]
    f_ref[...] = jnp.tanh(p[0] + p[1] + b_ref[...])


def _edge1_body(hood_ref, f1_ref, w11, b11, w21, b21, msg_ref):
    hood = hood_ref[...]
    h1 = jnp.tanh(hood[:, 0:1] * w11[0:1, :] + hood[:, 1:2] * w11[1:2, :]
                  + b11[...])
    k1 = jnp.dot(h1, w21[...], preferred_element_type=jnp.float32) + b21[...]
    s = jnp.sum(k1 * f1_ref[...], axis=1, keepdims=True)
    lane = lax.broadcasted_iota(jnp.int32, (s.shape[0], 8), 1)
    msg_ref[...] = jnp.where(lane == 0, s, 0.0)


def _pool_body(NB, MOLP, v0_ref, v1_ref, b1_ref, v2m_ref, molf_ref, moli_ref):
    bias = b1_ref[0, 0]
    mol_ids = lax.broadcasted_iota(jnp.int32, (MOLP, 128), 0)
    neg_inf = jnp.float32(-jnp.inf)
    int_min = jnp.int32(-2147483648)

    def pass1(k, cur):
        vals = v0_ref[pl.ds(k, 1), :] + v1_ref[pl.ds(k, 1), :] + bias
        seg = v2m_ref[pl.ds(k, 1), :]
        cand = jnp.where(seg == mol_ids, vals, neg_inf)
        return jnp.maximum(cur, jnp.max(cand, axis=1, keepdims=True))

    molmax = lax.fori_loop(0, NB, pass1,
                           jnp.full((MOLP, 1), neg_inf, jnp.float32))

    def pass2(k, cur):
        vals = v0_ref[pl.ds(k, 1), :] + v1_ref[pl.ds(k, 1), :] + bias
        seg = v2m_ref[pl.ds(k, 1), :]
        ids = lax.broadcasted_iota(jnp.int32, (MOLP, 128), 1) + k * 128
        mask = (seg == mol_ids) & (vals >= molmax)
        cand = jnp.where(mask, ids, int_min)
        return jnp.maximum(cur, jnp.max(cand, axis=1, keepdims=True))

    molidx = lax.fori_loop(0, NB, pass2,
                           jnp.full((MOLP, 1), int_min, jnp.int32))
    molf_ref[...] = molmax[0:NUM_MOL]
    moli_ref[...] = molidx[0:NUM_MOL]


def kernel(atom_features, atom_pos, molecule_edges, vertex2molecule,
           num_molecules, W1_0, b1_0, W2_0, b2_0, bias_0,
           W1_1, b1_1, W2_1, b2_1, bias_1):
    del num_molecules
    N, F = atom_features.shape
    E = molecule_edges.shape[0]
    f32, i32 = jnp.float32, jnp.int32

    NP = ((N + 255) // 256) * 256
    EPW = NW * CH
    EP = ((E + EPW - 1) // EPW) * EPW
    T = EP // (NW * CH)
    BE = 2048
    NBLK = NP // 128

    src = molecule_edges[:, 0]
    dst = molecule_edges[:, 1]
    padi = jnp.full((EP - E,), N, dtype=i32)
    srcp = jnp.concatenate([src, padi]).reshape(EP // 128, 128)
    dstp = jnp.concatenate([dst, padi]).reshape(EP // 128, 128)

    tsrc = jnp.zeros((NP, 16), f32)
    tsrc = lax.dynamic_update_slice(tsrc, atom_features, (0, 0))
    tsrc = lax.dynamic_update_slice(tsrc, atom_pos, (0, F))
    posd = jnp.zeros((NP, 8), f32)
    posd = lax.dynamic_update_slice(posd, atom_pos, (0, 0))

    zeros32 = jnp.zeros((NP, HID), f32)
    zeros1 = jnp.zeros((NP, 8), f32)

    w2p = W2_0.reshape(HID, HID, F).transpose(0, 2, 1).reshape(HID, HID * F)
    b0m = b2_0.reshape(HID, F).T
    b10 = b1_0.reshape(1, HID)
    b11 = b1_1.reshape(1, HID)
    b21 = b2_1.reshape(1, HID)
    b1s = bias_1.reshape(1, 1)
    v2m = jnp.concatenate(
        [vertex2molecule, jnp.full((NP - N,), -1, i32)]).reshape(NBLK, 128)

    gather_edges = pl.kernel(
        functools.partial(_gather_edges_body, T),
        out_type=[jax.ShapeDtypeStruct((EP, 16), f32),
                  jax.ShapeDtypeStruct((EP, 8), f32)],
        mesh=_sc_mesh(),
        scratch_types=[pltpu.VMEM((IR, 128), i32),
                       pltpu.VMEM((IR, 128), i32),
                       pltpu.VMEM((CH, 16), f32),
                       pltpu.VMEM((CH, 8), f32),
                       pltpu.SemaphoreType.DMA,
                       pltpu.SemaphoreType.DMA],
        compiler_params=_sc_params,
        name="sc_gather_edges")
    a_e, pd_e = gather_edges(tsrc, posd, srcp, dstp)

    nb = EP // BE
    msg0, hood = pl.pallas_call(
        _edge0_body,
        grid=(nb,),
        in_specs=[pl.BlockSpec((BE, 16), lambda i: (i, 0)),
                  pl.BlockSpec((BE, 8), lambda i: (i, 0)),
                  pl.BlockSpec((2, HID), lambda i: (0, 0)),
                  pl.BlockSpec((1, HID), lambda i: (0, 0)),
                  pl.BlockSpec((HID, HID * F), lambda i: (0, 0)),
                  pl.BlockSpec((F, HID), lambda i: (0, 0))],
        out_specs=[pl.BlockSpec((BE, HID), lambda i: (i, 0)),
                   pl.BlockSpec((BE, 2), lambda i: (i, 0))],
        out_shape=[jax.ShapeDtypeStruct((EP, HID), f32),
                   jax.ShapeDtypeStruct((EP, 2), f32)],
        name="tc_edge0")(a_e, pd_e, W1_0, b10, w2p, b0m)

    scatter32 = pl.kernel(
        functools.partial(_scatter_add_body, T, NP),
        out_type=jax.ShapeDtypeStruct((NC, NP, HID), f32),
        mesh=_sc_mesh(),
        scratch_types=[pltpu.VMEM((IR, 128), i32),
                       pltpu.VMEM((CH, HID), f32),
                       pltpu.VMEM_SHARED((NP, HID), f32),
                       pltpu.SemaphoreType.DMA],
        compiler_params=_sc_params,
        name="sc_scatter_msg0")
    agg0 = scatter32(msg0, dstp, zeros32)

    feats = pl.pallas_call(
        _node_tanh_body,
        in_specs=[pl.BlockSpec((NC, NP, HID), lambda: (0, 0, 0)),
                  pl.BlockSpec((1, HID), lambda: (0, 0))],
        out_specs=pl.BlockSpec((NP, HID), lambda: (0, 0)),
        out_shape=jax.ShapeDtypeStruct((NP, HID), f32),
        name="tc_node_tanh")(agg0, bias_0.reshape(1, HID))

    gather32 = pl.kernel(
        functools.partial(_gather_table_body, T),
        out_type=jax.ShapeDtypeStruct((EP, HID), f32),
        mesh=_sc_mesh(),
        scratch_types=[pltpu.VMEM((IR, 128), i32),
                       pltpu.VMEM((CH, HID), f32),
                       pltpu.SemaphoreType.DMA],
        compiler_params=_sc_params,
        name="sc_gather_feats")
    f1 = gather32(feats, srcp)

    msg1 = pl.pallas_call(
        _edge1_body,
        grid=(nb,),
        in_specs=[pl.BlockSpec((BE, 2), lambda i: (i, 0)),
                  pl.BlockSpec((BE, HID), lambda i: (i, 0)),
                  pl.BlockSpec((2, HID), lambda i: (0, 0)),
                  pl.BlockSpec((1, HID), lambda i: (0, 0)),
                  pl.BlockSpec((HID, HID), lambda i: (0, 0)),
                  pl.BlockSpec((1, HID), lambda i: (0, 0))],
        out_specs=pl.BlockSpec((BE, 8), lambda i: (i, 0)),
        out_shape=jax.ShapeDtypeStruct((EP, 8), f32),
        name="tc_edge1")(hood, f1, W1_1, b11, W2_1, b21)

    scatter1 = pl.kernel(
        functools.partial(_scatter_add_body, T, NP),
        out_type=jax.ShapeDtypeStruct((NC, NP, 8), f32),
        mesh=_sc_mesh(),
        scratch_types=[pltpu.VMEM((IR, 128), i32),
                       pltpu.VMEM((CH, 8), f32),
                       pltpu.VMEM_SHARED((NP, 8), f32),
                       pltpu.SemaphoreType.DMA],
        compiler_params=_sc_params,
        name="sc_scatter_msg1")
    agg1 = scatter1(msg1, dstp, zeros1)

    v0 = agg1[0, :, 0].reshape(NBLK, 128)
    v1 = agg1[1, :, 0].reshape(NBLK, 128)
    MOLP = 512
    molf, moli = pl.pallas_call(
        functools.partial(_pool_body, NBLK, MOLP),
        in_specs=[pl.BlockSpec((NBLK, 128), lambda: (0, 0)),
                  pl.BlockSpec((NBLK, 128), lambda: (0, 0)),
                  pl.BlockSpec((1, 1), lambda: (0, 0)),
                  pl.BlockSpec((NBLK, 128), lambda: (0, 0))],
        out_specs=[pl.BlockSpec((NUM_MOL, 1), lambda: (0, 0)),
                   pl.BlockSpec((NUM_MOL, 1), lambda: (0, 0))],
        out_shape=[jax.ShapeDtypeStruct((NUM_MOL, 1), f32),
                   jax.ShapeDtypeStruct((NUM_MOL, 1), i32)],
        name="tc_pool")(v0, v1, b1s, v2m)

    return (molf, molf, moli)


# speedup vs baseline: 4.0057x; 1.5056x over previous
import functools

import jax
import jax.numpy as jnp
from jax import lax
from jax.experimental import pallas as pl
from jax.experimental.pallas import tpu as pltpu
from jax.experimental.pallas import tpu_sc as plsc

RADIUS = 1.54
HID = 32
NUM_MOL = 500

NC, NS = 2, 16
NW = NC * NS
CH = 1024
IR = CH // 128

_sc_mesh = functools.partial(
    plsc.VectorSubcoreMesh, core_axis_name="c", subcore_axis_name="s",
    num_cores=NC, num_subcores=NS)
_sc_params = pltpu.CompilerParams(use_tc_tiling_on_sc=False)


def _worker_id():
    return lax.axis_index("s") * NC + lax.axis_index("c")


def _gather_edges_body(T, tsrc, srcp, dstp, a_out, d_out,
                       idx_s, idx_d, abuf, dbuf, sem, sem2):
    w = _worker_id()

    def chunk(t, carry):
        row0 = (w * T + t) * IR
        e0 = (w * T + t) * CH
        pltpu.sync_copy(srcp.at[pl.ds(row0, IR)], idx_s)
        pltpu.sync_copy(dstp.at[pl.ds(row0, IR)], idx_d)
        cps = [pltpu.async_copy(tsrc.at[idx_s.at[j]],
                                abuf.at[pl.ds(j * 128, 128)], sem)
               for j in range(IR)]
        cps += [pltpu.async_copy(tsrc.at[idx_d.at[j]],
                                 dbuf.at[pl.ds(j * 128, 128)], sem2)
                for j in range(IR)]
        for cp in cps:
            cp.wait()
        pltpu.sync_copy(abuf, a_out.at[pl.ds(e0, CH)])
        pltpu.sync_copy(dbuf, d_out.at[pl.ds(e0, CH)])
        return carry

    lax.fori_loop(0, T, chunk, 0)


def _gather_table_body(T, table, srcp, out, idx_s, buf, sem):
    w = _worker_id()

    def chunk(t, carry):
        row0 = (w * T + t) * IR
        e0 = (w * T + t) * CH
        pltpu.sync_copy(srcp.at[pl.ds(row0, IR)], idx_s)
        cps = [pltpu.async_copy(table.at[idx_s.at[j]],
                                buf.at[pl.ds(j * 128, 128)], sem)
               for j in range(IR)]
        for cp in cps:
            cp.wait()
        pltpu.sync_copy(buf, out.at[pl.ds(e0, CH)])
        return carry

    lax.fori_loop(0, T, chunk, 0)


def _scatter_add_body(T, NP, msg, dstp, zeros, out, idx_d, mbuf, table, sem):
    c = lax.axis_index("c")
    s = lax.axis_index("s")
    w = s * NC + c
    rpt = NP // NS
    pltpu.sync_copy(zeros.at[pl.ds(s * rpt, rpt)],
                    table.at[pl.ds(s * rpt, rpt)])
    plsc.subcore_barrier()

    def chunk(t, carry):
        row0 = (w * T + t) * IR
        e0 = (w * T + t) * CH
        pltpu.sync_copy(dstp.at[pl.ds(row0, IR)], idx_d)
        pltpu.sync_copy(msg.at[pl.ds(e0, CH)], mbuf)
        for j in range(IR):
            pltpu.sync_copy(mbuf.at[pl.ds(j * 128, 128)],
                            table.at[idx_d.at[j]], add=True)
        return carry

    lax.fori_loop(0, T, chunk, 0)
    plsc.subcore_barrier()
    pltpu.sync_copy(table.at[pl.ds(s * rpt, rpt)],
                    out.at[c].at[pl.ds(s * rpt, rpt)])


def _edge0_body(a_ref, d_ref, w10t, b10c, w11t, b11c, w21t, b21c,
                w2pt, r7t, s224t, b0mt, msg_ref, k1_ref):
    at = a_ref[...].T
    dt = d_ref[...].T
    for k in range(4):
        ak = at[k * HID:(k + 1) * HID, :]
        dk = dt[k * HID:(k + 1) * HID, :]
        hoodk = ak[7:9, :] - dk[7:9, :]
        h0 = jnp.tanh(jnp.dot(w10t[...], hoodk,
                              preferred_element_type=jnp.float32) + b10c[...])
        p = jnp.dot(w2pt[...], h0, preferred_element_type=jnp.float32)
        fk = ak[0:7, :]
        frep = jnp.dot(r7t[...], fk, preferred_element_type=jnp.float32)
        acc = (jnp.dot(s224t[...], p * frep,
                       preferred_element_type=jnp.float32)
               + jnp.dot(b0mt[...], fk, preferred_element_type=jnp.float32))
        h1 = jnp.tanh(jnp.dot(w11t[...], hoodk,
                              preferred_element_type=jnp.float32) + b11c[...])
        k1 = jnp.dot(w21t[...], h1,
                     preferred_element_type=jnp.float32) + b21c[...]
        msg_ref[:, k * HID:(k + 1) * HID] = acc.T
        k1_ref[:, k * HID:(k + 1) * HID] = k1.T


def _node_tanh_body(p_ref, b_ref, f_ref):
    p = p_ref[...]
    f_ref[...] = jnp.tanh(p[0] + p[1] + b_ref[...])


def _edge1_body(k1_ref, f1_ref, z4, msg_ref):
    msg_ref[...] = jnp.dot(k1_ref[...] * f1_ref[...], z4[...],
                           preferred_element_type=jnp.float32)


def _pool_body(NB, MOLP, v0_ref, v1_ref, b1_ref, v2m_ref, molf_ref, moli_ref):
    bias = b1_ref[0, 0]
    mol_ids = lax.broadcasted_iota(jnp.int32, (MOLP, 128), 0)
    neg_inf = jnp.float32(-jnp.inf)
    int_min = jnp.int32(-2147483648)

    def pass1(k, cur):
        vals = v0_ref[pl.ds(k, 1), :] + v1_ref[pl.ds(k, 1), :] + bias
        seg = v2m_ref[pl.ds(k, 1), :]
        cand = jnp.where(seg == mol_ids, vals, neg_inf)
        return jnp.maximum(cur, jnp.max(cand, axis=1, keepdims=True))

    molmax = lax.fori_loop(0, NB, pass1,
                           jnp.full((MOLP, 1), neg_inf, jnp.float32))

    def pass2(k, cur):
        vals = v0_ref[pl.ds(k, 1), :] + v1_ref[pl.ds(k, 1), :] + bias
        seg = v2m_ref[pl.ds(k, 1), :]
        ids = lax.broadcasted_iota(jnp.int32, (MOLP, 128), 1) + k * 128
        mask = (seg == mol_ids) & (vals >= molmax)
        cand = jnp.where(mask, ids, int_min)
        return jnp.maximum(cur, jnp.max(cand, axis=1, keepdims=True))

    molidx = lax.fori_loop(0, NB, pass2,
                           jnp.full((MOLP, 1), int_min, jnp.int32))
    molf_ref[...] = molmax[0:NUM_MOL]
    moli_ref[...] = molidx[0:NUM_MOL]


def kernel(atom_features, atom_pos, molecule_edges, vertex2molecule,
           num_molecules, W1_0, b1_0, W2_0, b2_0, bias_0,
           W1_1, b1_1, W2_1, b2_1, bias_1):
    del num_molecules
    N, F = atom_features.shape
    E = molecule_edges.shape[0]
    f32, i32 = jnp.float32, jnp.int32

    NP = ((N + 255) // 256) * 256
    EPW = NW * CH
    EP = ((E + EPW - 1) // EPW) * EPW
    T = EP // (NW * CH)
    BE = 2048
    NBLK = NP // 128

    src = molecule_edges[:, 0]
    dst = molecule_edges[:, 1]
    padi = jnp.full((EP - E,), N, dtype=i32)
    srcp = jnp.concatenate([src, padi]).reshape(EP // 128, 128)
    dstp = jnp.concatenate([dst, padi]).reshape(EP // 128, 128)

    tsrc = jnp.zeros((NP, HID), f32)
    tsrc = lax.dynamic_update_slice(tsrc, atom_features, (0, 0))
    tsrc = lax.dynamic_update_slice(tsrc, atom_pos, (0, F))

    zeros32 = jnp.zeros((NP, HID), f32)

    inv_r = 1.0 / RADIUS
    w10t = (W1_0 * inv_r).T
    w11t = (W1_1 * inv_r).T
    b10c = b1_0.reshape(HID, 1)
    b11c = b1_1.reshape(HID, 1)
    b21c = b2_1.reshape(HID, 1)
    w21t = W2_1.T
    w2pt = W2_0.reshape(HID, HID, F).transpose(2, 1, 0).reshape(HID * F, HID)
    r7t = jnp.kron(jnp.eye(F, dtype=f32), jnp.ones((HID, 1), f32))
    s224t = jnp.tile(jnp.eye(HID, dtype=f32), (1, F))
    b0mt = b2_0.reshape(HID, F)
    zpat = jnp.zeros((HID, HID), f32).at[:, 0].set(1.0)
    z4 = jnp.kron(jnp.eye(4, dtype=f32), zpat)
    b1s = bias_1.reshape(1, 1)
    v2m = jnp.concatenate(
        [vertex2molecule, jnp.full((NP - N,), -1, i32)]).reshape(NBLK, 128)

    gather_edges = pl.kernel(
        functools.partial(_gather_edges_body, T),
        out_type=[jax.ShapeDtypeStruct((EP, HID), f32),
                  jax.ShapeDtypeStruct((EP, HID), f32)],
        mesh=_sc_mesh(),
        scratch_types=[pltpu.VMEM((IR, 128), i32),
                       pltpu.VMEM((IR, 128), i32),
                       pltpu.VMEM((CH, HID), f32),
                       pltpu.VMEM((CH, HID), f32),
                       pltpu.SemaphoreType.DMA,
                       pltpu.SemaphoreType.DMA],
        compiler_params=_sc_params,
        name="sc_gather_edges")
    a_e, d_e = gather_edges(tsrc, srcp, dstp)

    nb = EP // BE
    BR = BE // 4
    a4 = a_e.reshape(EP // 4, 128)
    d4 = d_e.reshape(EP // 4, 128)
    const_spec = lambda r, c: pl.BlockSpec((r, c), lambda i: (0, 0))
    msg0, k1e = pl.pallas_call(
        _edge0_body,
        grid=(nb,),
        in_specs=[pl.BlockSpec((BR, 128), lambda i: (i, 0)),
                  pl.BlockSpec((BR, 128), lambda i: (i, 0)),
                  const_spec(HID, 2), const_spec(HID, 1),
                  const_spec(HID, 2), const_spec(HID, 1),
                  const_spec(HID, HID), const_spec(HID, 1),
                  const_spec(HID * F, HID), const_spec(HID * F, F),
                  const_spec(HID, HID * F), const_spec(HID, F)],
        out_specs=[pl.BlockSpec((BR, 128), lambda i: (i, 0)),
                   pl.BlockSpec((BR, 128), lambda i: (i, 0))],
        out_shape=[jax.ShapeDtypeStruct((EP // 4, 128), f32),
                   jax.ShapeDtypeStruct((EP // 4, 128), f32)],
        name="tc_edge0")(a4, d4, w10t, b10c, w11t, b11c, w21t, b21c,
                         w2pt, r7t, s224t, b0mt)

    scatter32 = pl.kernel(
        functools.partial(_scatter_add_body, T, NP),
        out_type=jax.ShapeDtypeStruct((NC, NP, HID), f32),
        mesh=_sc_mesh(),
        scratch_types=[pltpu.VMEM((IR, 128), i32),
                       pltpu.VMEM((CH, HID), f32),
                       pltpu.VMEM_SHARED((NP, HID), f32),
                       pltpu.SemaphoreType.DMA],
        compiler_params=_sc_params,
        name="sc_scatter_msg0")
    agg0 = scatter32(msg0.reshape(EP, HID), dstp, zeros32)

    feats = pl.pallas_call(
        _node_tanh_body,
        in_specs=[pl.BlockSpec((NC, NP, HID), lambda: (0, 0, 0)),
                  pl.BlockSpec((1, HID), lambda: (0, 0))],
        out_specs=pl.BlockSpec((NP, HID), lambda: (0, 0)),
        out_shape=jax.ShapeDtypeStruct((NP, HID), f32),
        name="tc_node_tanh")(agg0, bias_0.reshape(1, HID))

    gather32 = pl.kernel(
        functools.partial(_gather_table_body, T),
        out_type=jax.ShapeDtypeStruct((EP, HID), f32),
        mesh=_sc_mesh(),
        scratch_types=[pltpu.VMEM((IR, 128), i32),
                       pltpu.VMEM((CH, HID), f32),
                       pltpu.SemaphoreType.DMA],
        compiler_params=_sc_params,
        name="sc_gather_feats")
    f1 = gather32(feats, srcp)

    msg1 = pl.pallas_call(
        _edge1_body,
        grid=(nb,),
        in_specs=[pl.BlockSpec((BR, 128), lambda i: (i, 0)),
                  pl.BlockSpec((BR, 128), lambda i: (i, 0)),
                  const_spec(128, 128)],
        out_specs=pl.BlockSpec((BR, 128), lambda i: (i, 0)),
        out_shape=jax.ShapeDtypeStruct((EP // 4, 128), f32),
        name="tc_edge1")(k1e, f1.reshape(EP // 4, 128), z4)

    agg1 = scatter32(msg1.reshape(EP, HID), dstp, zeros32)

    v0 = agg1[0, :, 0].reshape(NBLK, 128)
    v1 = agg1[1, :, 0].reshape(NBLK, 128)
    MOLP = 512
    molf, moli = pl.pallas_call(
        functools.partial(_pool_body, NBLK, MOLP),
        in_specs=[pl.BlockSpec((NBLK, 128), lambda: (0, 0)),
                  pl.BlockSpec((NBLK, 128), lambda: (0, 0)),
                  pl.BlockSpec((1, 1), lambda: (0, 0)),
                  pl.BlockSpec((NBLK, 128), lambda: (0, 0))],
        out_specs=[pl.BlockSpec((NUM_MOL, 1), lambda: (0, 0)),
                   pl.BlockSpec((NUM_MOL, 1), lambda: (0, 0))],
        out_shape=[jax.ShapeDtypeStruct((NUM_MOL, 1), f32),
                   jax.ShapeDtypeStruct((NUM_MOL, 1), jnp.int32)],
        name="tc_pool")(v0, v1, b1s, v2m)

    return (molf, molf, moli)


# speedup vs baseline: 4.7784x; 1.1929x over previous
import functools

import jax
import jax.numpy as jnp
from jax import lax
from jax.experimental import pallas as pl
from jax.experimental.pallas import tpu as pltpu
from jax.experimental.pallas import tpu_sc as plsc

RADIUS = 1.54
HID = 32
NUM_MOL = 500

NC, NS = 2, 16
NW = NC * NS
CH = 1024
IR = CH // 128

_sc_mesh = functools.partial(
    plsc.VectorSubcoreMesh, core_axis_name="c", subcore_axis_name="s",
    num_cores=NC, num_subcores=NS)
_sc_params = pltpu.CompilerParams(use_tc_tiling_on_sc=False)


def _worker_id():
    return lax.axis_index("s") * NC + lax.axis_index("c")


def _gather_edges_body(T, tsrc, srcp, dstp, a_out, d_out,
                       idx_s, idx_d, abuf, dbuf, sem, sem2):
    w = _worker_id()

    def chunk(t, carry):
        row0 = (w * T + t) * IR
        e0 = (w * T + t) * CH
        pltpu.sync_copy(srcp.at[pl.ds(row0, IR)], idx_s)
        pltpu.sync_copy(dstp.at[pl.ds(row0, IR)], idx_d)
        cps = [pltpu.async_copy(tsrc.at[idx_s.at[j]],
                                abuf.at[pl.ds(j * 128, 128)], sem)
               for j in range(IR)]
        cps += [pltpu.async_copy(tsrc.at[idx_d.at[j]],
                                 dbuf.at[pl.ds(j * 128, 128)], sem2)
                for j in range(IR)]
        for cp in cps:
            cp.wait()
        pltpu.sync_copy(abuf, a_out.at[pl.ds(e0, CH)])
        pltpu.sync_copy(dbuf, d_out.at[pl.ds(e0, CH)])
        return carry

    lax.fori_loop(0, T, chunk, 0)


def _gather_table_body(T, table, srcp, out, idx_s, buf, sem):
    w = _worker_id()

    def chunk(t, carry):
        row0 = (w * T + t) * IR
        e0 = (w * T + t) * CH
        pltpu.sync_copy(srcp.at[pl.ds(row0, IR)], idx_s)
        cps = [pltpu.async_copy(table.at[idx_s.at[j]],
                                buf.at[pl.ds(j * 128, 128)], sem)
               for j in range(IR)]
        for cp in cps:
            cp.wait()
        pltpu.sync_copy(buf, out.at[pl.ds(e0, CH)])
        return carry

    lax.fori_loop(0, T, chunk, 0)


def _scatter_add_body(T, NP, msg, dstp, zeros, out, idx_d, mbuf, table, sem):
    c = lax.axis_index("c")
    s = lax.axis_index("s")
    w = s * NC + c
    rpt = NP // NS
    pltpu.sync_copy(zeros.at[pl.ds(s * rpt, rpt)],
                    table.at[pl.ds(s * rpt, rpt)])
    plsc.subcore_barrier()

    def chunk(t, carry):
        row0 = (w * T + t) * IR
        e0 = (w * T + t) * CH
        pltpu.sync_copy(dstp.at[pl.ds(row0, IR)], idx_d)
        pltpu.sync_copy(msg.at[pl.ds(e0, CH)], mbuf)
        for j in range(IR):
            pltpu.sync_copy(mbuf.at[pl.ds(j * 128, 128)],
                            table.at[idx_d.at[j]], add=True)
        return carry

    lax.fori_loop(0, T, chunk, 0)
    plsc.subcore_barrier()
    pltpu.sync_copy(table.at[pl.ds(s * rpt, rpt)],
                    out.at[c].at[pl.ds(s * rpt, rpt)])


def _edge0_body(a_ref, d_ref, w10, b10r, w11, b11r, w21, b21r,
                w2p, r7, s224, b0m, msg_ref, k1_ref):
    a = a_ref[...]
    d = d_ref[...]
    for k in range(4):
        ak = a[:, k * HID:(k + 1) * HID]
        dk = d[:, k * HID:(k + 1) * HID]
        hoodk = ak[:, 7:9] - dk[:, 7:9]
        h0 = jnp.tanh(jnp.dot(hoodk, w10[...],
                              preferred_element_type=jnp.float32) + b10r[...])
        p = jnp.dot(h0, w2p[...], preferred_element_type=jnp.float32)
        fk = ak[:, 0:7]
        frep = jnp.dot(fk, r7[...], preferred_element_type=jnp.float32)
        acc = (jnp.dot(p * frep, s224[...],
                       preferred_element_type=jnp.float32)
               + jnp.dot(fk, b0m[...], preferred_element_type=jnp.float32))
        h1 = jnp.tanh(jnp.dot(hoodk, w11[...],
                              preferred_element_type=jnp.float32) + b11r[...])
        k1 = jnp.dot(h1, w21[...],
                     preferred_element_type=jnp.float32) + b21r[...]
        msg_ref[:, k * HID:(k + 1) * HID] = acc
        k1_ref[:, k * HID:(k + 1) * HID] = k1


def _node_tanh_body(p_ref, b_ref, f_ref):
    p = p_ref[...]
    f_ref[...] = jnp.tanh(p[0] + p[1] + b_ref[...])


def _edge1_body(k1_ref, f1_ref, z4, msg_ref):
    msg_ref[...] = jnp.dot(k1_ref[...] * f1_ref[...], z4[...],
                           preferred_element_type=jnp.float32)


def _pool_body(NB, MOLP, v0_ref, v1_ref, b1_ref, v2m_ref, molf_ref, moli_ref):
    bias = b1_ref[0, 0]
    mol_ids = lax.broadcasted_iota(jnp.int32, (MOLP, 128), 0)
    neg_inf = jnp.float32(-jnp.inf)
    int_min = jnp.int32(-2147483648)

    def pass1(k, cur):
        vals = v0_ref[pl.ds(k, 1), :] + v1_ref[pl.ds(k, 1), :] + bias
        seg = v2m_ref[pl.ds(k, 1), :]
        cand = jnp.where(seg == mol_ids, vals, neg_inf)
        return jnp.maximum(cur, jnp.max(cand, axis=1, keepdims=True))

    molmax = lax.fori_loop(0, NB, pass1,
                           jnp.full((MOLP, 1), neg_inf, jnp.float32))

    def pass2(k, cur):
        vals = v0_ref[pl.ds(k, 1), :] + v1_ref[pl.ds(k, 1), :] + bias
        seg = v2m_ref[pl.ds(k, 1), :]
        ids = lax.broadcasted_iota(jnp.int32, (MOLP, 128), 1) + k * 128
        mask = (seg == mol_ids) & (vals >= molmax)
        cand = jnp.where(mask, ids, int_min)
        return jnp.maximum(cur, jnp.max(cand, axis=1, keepdims=True))

    molidx = lax.fori_loop(0, NB, pass2,
                           jnp.full((MOLP, 1), int_min, jnp.int32))
    molf_ref[...] = molmax[0:NUM_MOL]
    moli_ref[...] = molidx[0:NUM_MOL]


def kernel(atom_features, atom_pos, molecule_edges, vertex2molecule,
           num_molecules, W1_0, b1_0, W2_0, b2_0, bias_0,
           W1_1, b1_1, W2_1, b2_1, bias_1):
    del num_molecules
    N, F = atom_features.shape
    E = molecule_edges.shape[0]
    f32, i32 = jnp.float32, jnp.int32

    NP = ((N + 255) // 256) * 256
    EPW = NW * CH
    EP = ((E + EPW - 1) // EPW) * EPW
    T = EP // (NW * CH)
    BE = 2048
    NBLK = NP // 128

    src = molecule_edges[:, 0]
    dst = molecule_edges[:, 1]
    padi = jnp.full((EP - E,), N, dtype=i32)
    srcp = jnp.concatenate([src, padi]).reshape(EP // 128, 128)
    dstp = jnp.concatenate([dst, padi]).reshape(EP // 128, 128)

    tsrc = jnp.pad(jnp.concatenate([atom_features, atom_pos], axis=1),
                   ((0, NP - N), (0, HID - F - 2)))

    zeros32 = jnp.zeros((NP, HID), f32)

    inv_r = 1.0 / RADIUS
    w10 = W1_0 * inv_r
    w11 = W1_1 * inv_r
    b10r = b1_0.reshape(1, HID)
    b11r = b1_1.reshape(1, HID)
    b21r = b2_1.reshape(1, HID)
    w21 = W2_1
    w2p = W2_0.reshape(HID, HID, F).transpose(0, 2, 1).reshape(HID, HID * F)
    r7 = jnp.kron(jnp.eye(F, dtype=f32), jnp.ones((1, HID), f32))
    s224 = jnp.tile(jnp.eye(HID, dtype=f32), (F, 1))
    b0m = b2_0.reshape(HID, F).T
    zpat = jnp.zeros((HID, HID), f32).at[:, 0].set(1.0)
    z4 = jnp.kron(jnp.eye(4, dtype=f32), zpat)
    b1s = bias_1.reshape(1, 1)
    v2m = jnp.concatenate(
        [vertex2molecule, jnp.full((NP - N,), -1, i32)]).reshape(NBLK, 128)

    gather_edges = pl.kernel(
        functools.partial(_gather_edges_body, T),
        out_type=[jax.ShapeDtypeStruct((EP, HID), f32),
                  jax.ShapeDtypeStruct((EP, HID), f32)],
        mesh=_sc_mesh(),
        scratch_types=[pltpu.VMEM((IR, 128), i32),
                       pltpu.VMEM((IR, 128), i32),
                       pltpu.VMEM((CH, HID), f32),
                       pltpu.VMEM((CH, HID), f32),
                       pltpu.SemaphoreType.DMA,
                       pltpu.SemaphoreType.DMA],
        compiler_params=_sc_params,
        name="sc_gather_edges")
    a_e, d_e = gather_edges(tsrc, srcp, dstp)

    nb = EP // BE
    BR = BE // 4
    a4 = a_e.reshape(EP // 4, 128)
    d4 = d_e.reshape(EP // 4, 128)
    const_spec = lambda r, c: pl.BlockSpec((r, c), lambda i: (0, 0))
    msg0, k1e = pl.pallas_call(
        _edge0_body,
        grid=(nb,),
        in_specs=[pl.BlockSpec((BR, 128), lambda i: (i, 0)),
                  pl.BlockSpec((BR, 128), lambda i: (i, 0)),
                  const_spec(2, HID), const_spec(1, HID),
                  const_spec(2, HID), const_spec(1, HID),
                  const_spec(HID, HID), const_spec(1, HID),
                  const_spec(HID, HID * F), const_spec(F, HID * F),
                  const_spec(HID * F, HID), const_spec(F, HID)],
        out_specs=[pl.BlockSpec((BR, 128), lambda i: (i, 0)),
                   pl.BlockSpec((BR, 128), lambda i: (i, 0))],
        out_shape=[jax.ShapeDtypeStruct((EP // 4, 128), f32),
                   jax.ShapeDtypeStruct((EP // 4, 128), f32)],
        name="tc_edge0")(a4, d4, w10, b10r, w11, b11r, w21, b21r,
                         w2p, r7, s224, b0m)

    scatter32 = pl.kernel(
        functools.partial(_scatter_add_body, T, NP),
        out_type=jax.ShapeDtypeStruct((NC, NP, HID), f32),
        mesh=_sc_mesh(),
        scratch_types=[pltpu.VMEM((IR, 128), i32),
                       pltpu.VMEM((CH, HID), f32),
                       pltpu.VMEM_SHARED((NP, HID), f32),
                       pltpu.SemaphoreType.DMA],
        compiler_params=_sc_params,
        name="sc_scatter_msg0")
    agg0 = scatter32(msg0.reshape(EP, HID), dstp, zeros32)

    feats = pl.pallas_call(
        _node_tanh_body,
        in_specs=[pl.BlockSpec((NC, NP, HID), lambda: (0, 0, 0)),
                  pl.BlockSpec((1, HID), lambda: (0, 0))],
        out_specs=pl.BlockSpec((NP, HID), lambda: (0, 0)),
        out_shape=jax.ShapeDtypeStruct((NP, HID), f32),
        name="tc_node_tanh")(agg0, bias_0.reshape(1, HID))

    gather32 = pl.kernel(
        functools.partial(_gather_table_body, T),
        out_type=jax.ShapeDtypeStruct((EP, HID), f32),
        mesh=_sc_mesh(),
        scratch_types=[pltpu.VMEM((IR, 128), i32),
                       pltpu.VMEM((CH, HID), f32),
                       pltpu.SemaphoreType.DMA],
        compiler_params=_sc_params,
        name="sc_gather_feats")
    f1 = gather32(feats, srcp)

    msg1 = pl.pallas_call(
        _edge1_body,
        grid=(nb,),
        in_specs=[pl.BlockSpec((BR, 128), lambda i: (i, 0)),
                  pl.BlockSpec((BR, 128), lambda i: (i, 0)),
                  const_spec(128, 128)],
        out_specs=pl.BlockSpec((BR, 128), lambda i: (i, 0)),
        out_shape=jax.ShapeDtypeStruct((EP // 4, 128), f32),
        name="tc_edge1")(k1e, f1.reshape(EP // 4, 128), z4)

    agg1 = scatter32(msg1.reshape(EP, HID), dstp, zeros32)

    v0 = agg1[0, :, 0].reshape(NBLK, 128)
    v1 = agg1[1, :, 0].reshape(NBLK, 128)
    MOLP = 512
    molf, moli = pl.pallas_call(
        functools.partial(_pool_body, NBLK, MOLP),
        in_specs=[pl.BlockSpec((NBLK, 128), lambda: (0, 0)),
                  pl.BlockSpec((NBLK, 128), lambda: (0, 0)),
                  pl.BlockSpec((1, 1), lambda: (0, 0)),
                  pl.BlockSpec((NBLK, 128), lambda: (0, 0))],
        out_specs=[pl.BlockSpec((NUM_MOL, 1), lambda: (0, 0)),
                   pl.BlockSpec((NUM_MOL, 1), lambda: (0, 0))],
        out_shape=[jax.ShapeDtypeStruct((NUM_MOL, 1), f32),
                   jax.ShapeDtypeStruct((NUM_MOL, 1), jnp.int32)],
        name="tc_pool")(v0, v1, b1s, v2m)

    return (molf, molf, moli)


# speedup vs baseline: 5.5132x; 1.1538x over previous
import functools

import jax
import jax.numpy as jnp
from jax import lax
from jax.experimental import pallas as pl
from jax.experimental.pallas import tpu as pltpu
from jax.experimental.pallas import tpu_sc as plsc

RADIUS = 1.54
HID = 32
NUM_MOL = 500

NC, NS = 2, 16
NW = NC * NS
CH = 1024
IR = CH // 128

_sc_mesh = functools.partial(
    plsc.VectorSubcoreMesh, core_axis_name="c", subcore_axis_name="s",
    num_cores=NC, num_subcores=NS)
_sc_params = pltpu.CompilerParams(use_tc_tiling_on_sc=False)


def _worker_id():
    return lax.axis_index("s") * NC + lax.axis_index("c")


def _gather_edges_body(T, tsrc, srcp, dstp, a_out, d_out,
                       idx_s, idx_d, abuf, dbuf, sem, sem2):
    w = _worker_id()

    def chunk(t, carry):
        row0 = (w * T + t) * IR
        e0 = (w * T + t) * CH
        pltpu.sync_copy(srcp.at[pl.ds(row0, IR)], idx_s)
        pltpu.sync_copy(dstp.at[pl.ds(row0, IR)], idx_d)
        cps = [pltpu.async_copy(tsrc.at[idx_s.at[j]],
                                abuf.at[pl.ds(j * 128, 128)], sem)
               for j in range(IR)]
        cps += [pltpu.async_copy(tsrc.at[idx_d.at[j]],
                                 dbuf.at[pl.ds(j * 128, 128)], sem2)
                for j in range(IR)]
        for cp in cps:
            cp.wait()
        pltpu.sync_copy(abuf, a_out.at[pl.ds(e0, CH)])
        pltpu.sync_copy(dbuf, d_out.at[pl.ds(e0, CH)])
        return carry

    lax.fori_loop(0, T, chunk, 0)


def _gather_table_body(T, table, srcp, out, idx_s, buf, sem):
    w = _worker_id()

    def chunk(t, carry):
        row0 = (w * T + t) * IR
        e0 = (w * T + t) * CH
        pltpu.sync_copy(srcp.at[pl.ds(row0, IR)], idx_s)
        cps = [pltpu.async_copy(table.at[idx_s.at[j]],
                                buf.at[pl.ds(j * 128, 128)], sem)
               for j in range(IR)]
        for cp in cps:
            cp.wait()
        pltpu.sync_copy(buf, out.at[pl.ds(e0, CH)])
        return carry

    lax.fori_loop(0, T, chunk, 0)


def _scatter_add_body(T, NP, msg, dstp, zeros, out, idx_d, mbuf, table, sem):
    c = lax.axis_index("c")
    s = lax.axis_index("s")
    w = s * NC + c
    rpt = NP // NS
    pltpu.sync_copy(zeros.at[pl.ds(s * rpt, rpt)],
                    table.at[pl.ds(s * rpt, rpt)])
    plsc.subcore_barrier()

    def chunk(t, carry):
        row0 = (w * T + t) * IR
        e0 = (w * T + t) * CH
        pltpu.sync_copy(dstp.at[pl.ds(row0, IR)], idx_d)
        pltpu.sync_copy(msg.at[pl.ds(e0, CH)], mbuf)
        for j in range(IR):
            pltpu.sync_copy(mbuf.at[pl.ds(j * 128, 128)],
                            table.at[idx_d.at[j]], add=True)
        return carry

    lax.fori_loop(0, T, chunk, 0)
    plsc.subcore_barrier()
    pltpu.sync_copy(table.at[pl.ds(s * rpt, rpt)],
                    out.at[c].at[pl.ds(s * rpt, rpt)])


def _edge0_body(a_ref, d_ref, w10, b10r, w11, b11r, w21, b21r,
                w2p, r7, s224, b0m, msg_ref, k1_ref):
    a = a_ref[...]
    d = d_ref[...]
    for k in range(4):
        ak = a[:, k * HID:(k + 1) * HID]
        dk = d[:, k * HID:(k + 1) * HID]
        hoodk = ak[:, 7:9] - dk[:, 7:9]
        h0 = jnp.tanh(jnp.dot(hoodk, w10[...],
                              preferred_element_type=jnp.float32) + b10r[...])
        p = jnp.dot(h0, w2p[...], preferred_element_type=jnp.float32)
        fk = ak[:, 0:7]
        frep = jnp.dot(fk, r7[...], preferred_element_type=jnp.float32)
        acc = (jnp.dot(p * frep, s224[...],
                       preferred_element_type=jnp.float32)
               + jnp.dot(fk, b0m[...], preferred_element_type=jnp.float32))
        h1 = jnp.tanh(jnp.dot(hoodk, w11[...],
                              preferred_element_type=jnp.float32) + b11r[...])
        k1 = jnp.dot(h1, w21[...],
                     preferred_element_type=jnp.float32) + b21r[...]
        msg_ref[:, k * HID:(k + 1) * HID] = acc
        k1_ref[:, k * HID:(k + 1) * HID] = k1


def _node_tanh_body(p_ref, b_ref, f_ref):
    p = p_ref[...]
    f_ref[...] = jnp.tanh(p[0] + p[1] + b_ref[...])


def _edge1_body(k1_ref, f1_ref, z4, msg_ref):
    msg_ref[...] = jnp.dot(k1_ref[...] * f1_ref[...], z4[...],
                           preferred_element_type=jnp.float32)


def _pool_body(NB, MOLP, v0_ref, v1_ref, b1_ref, v2m_ref, molf_ref, moli_ref):
    bias = b1_ref[0, 0]
    mol_ids = lax.broadcasted_iota(jnp.int32, (MOLP, 128), 0)
    neg_inf = jnp.float32(-jnp.inf)
    int_min = jnp.int32(-2147483648)

    def pass1(k, cur):
        vals = v0_ref[pl.ds(k, 1), :] + v1_ref[pl.ds(k, 1), :] + bias
        seg = v2m_ref[pl.ds(k, 1), :]
        cand = jnp.where(seg == mol_ids, vals, neg_inf)
        return jnp.maximum(cur, jnp.max(cand, axis=1, keepdims=True))

    molmax = lax.fori_loop(0, NB, pass1,
                           jnp.full((MOLP, 1), neg_inf, jnp.float32))

    def pass2(k, cur):
        vals = v0_ref[pl.ds(k, 1), :] + v1_ref[pl.ds(k, 1), :] + bias
        seg = v2m_ref[pl.ds(k, 1), :]
        ids = lax.broadcasted_iota(jnp.int32, (MOLP, 128), 1) + k * 128
        mask = (seg == mol_ids) & (vals >= molmax)
        cand = jnp.where(mask, ids, int_min)
        return jnp.maximum(cur, jnp.max(cand, axis=1, keepdims=True))

    molidx = lax.fori_loop(0, NB, pass2,
                           jnp.full((MOLP, 1), int_min, jnp.int32))
    molf_ref[...] = molmax[0:NUM_MOL]
    moli_ref[...] = molidx[0:NUM_MOL]


def kernel(atom_features, atom_pos, molecule_edges, vertex2molecule,
           num_molecules, W1_0, b1_0, W2_0, b2_0, bias_0,
           W1_1, b1_1, W2_1, b2_1, bias_1):
    del num_molecules
    N, F = atom_features.shape
    E = molecule_edges.shape[0]
    f32, i32 = jnp.float32, jnp.int32

    NP = ((N + 255) // 256) * 256
    EPW = NW * CH
    EP = ((E + EPW - 1) // EPW) * EPW
    T = EP // (NW * CH)
    BE = 8192
    NBLK = NP // 128

    src = molecule_edges[:, 0]
    dst = molecule_edges[:, 1]
    padi = jnp.full((EP - E,), N, dtype=i32)
    srcp = jnp.concatenate([src, padi]).reshape(EP // 128, 128)
    dstp = jnp.concatenate([dst, padi]).reshape(EP // 128, 128)

    tsrc = jnp.pad(jnp.concatenate([atom_features, atom_pos], axis=1),
                   ((0, NP - N), (0, HID - F - 2)))

    zeros32 = jnp.zeros((NP, HID), f32)

    inv_r = 1.0 / RADIUS
    w10 = W1_0 * inv_r
    w11 = W1_1 * inv_r
    b10r = b1_0.reshape(1, HID)
    b11r = b1_1.reshape(1, HID)
    b21r = b2_1.reshape(1, HID)
    w21 = W2_1
    w2p = W2_0.reshape(HID, HID, F).transpose(0, 2, 1).reshape(HID, HID * F)
    r7 = jnp.kron(jnp.eye(F, dtype=f32), jnp.ones((1, HID), f32))
    s224 = jnp.tile(jnp.eye(HID, dtype=f32), (F, 1))
    b0m = b2_0.reshape(HID, F).T
    zpat = jnp.zeros((HID, HID), f32).at[:, 0].set(1.0)
    z4 = jnp.kron(jnp.eye(4, dtype=f32), zpat)
    b1s = bias_1.reshape(1, 1)
    v2m = jnp.concatenate(
        [vertex2molecule, jnp.full((NP - N,), -1, i32)]).reshape(NBLK, 128)

    gather_edges = pl.kernel(
        functools.partial(_gather_edges_body, T),
        out_type=[jax.ShapeDtypeStruct((EP, HID), f32),
                  jax.ShapeDtypeStruct((EP, HID), f32)],
        mesh=_sc_mesh(),
        scratch_types=[pltpu.VMEM((IR, 128), i32),
                       pltpu.VMEM((IR, 128), i32),
                       pltpu.VMEM((CH, HID), f32),
                       pltpu.VMEM((CH, HID), f32),
                       pltpu.SemaphoreType.DMA,
                       pltpu.SemaphoreType.DMA],
        compiler_params=_sc_params,
        name="sc_gather_edges")
    a_e, d_e = gather_edges(tsrc, srcp, dstp)

    nb = EP // BE
    BR = BE // 4
    a4 = a_e.reshape(EP // 4, 128)
    d4 = d_e.reshape(EP // 4, 128)
    const_spec = lambda r, c: pl.BlockSpec((r, c), lambda i: (0, 0))
    msg0, k1e = pl.pallas_call(
        _edge0_body,
        grid=(nb,),
        in_specs=[pl.BlockSpec((BR, 128), lambda i: (i, 0)),
                  pl.BlockSpec((BR, 128), lambda i: (i, 0)),
                  const_spec(2, HID), const_spec(1, HID),
                  const_spec(2, HID), const_spec(1, HID),
                  const_spec(HID, HID), const_spec(1, HID),
                  const_spec(HID, HID * F), const_spec(F, HID * F),
                  const_spec(HID * F, HID), const_spec(F, HID)],
        out_specs=[pl.BlockSpec((BR, 128), lambda i: (i, 0)),
                   pl.BlockSpec((BR, 128), lambda i: (i, 0))],
        out_shape=[jax.ShapeDtypeStruct((EP // 4, 128), f32),
                   jax.ShapeDtypeStruct((EP // 4, 128), f32)],
        name="tc_edge0")(a4, d4, w10, b10r, w11, b11r, w21, b21r,
                         w2p, r7, s224, b0m)

    scatter32 = pl.kernel(
        functools.partial(_scatter_add_body, T, NP),
        out_type=jax.ShapeDtypeStruct((NC, NP, HID), f32),
        mesh=_sc_mesh(),
        scratch_types=[pltpu.VMEM((IR, 128), i32),
                       pltpu.VMEM((CH, HID), f32),
                       pltpu.VMEM_SHARED((NP, HID), f32),
                       pltpu.SemaphoreType.DMA],
        compiler_params=_sc_params,
        name="sc_scatter_msg0")
    agg0 = scatter32(msg0.reshape(EP, HID), dstp, zeros32)

    feats = pl.pallas_call(
        _node_tanh_body,
        in_specs=[pl.BlockSpec((NC, NP, HID), lambda: (0, 0, 0)),
                  pl.BlockSpec((1, HID), lambda: (0, 0))],
        out_specs=pl.BlockSpec((NP, HID), lambda: (0, 0)),
        out_shape=jax.ShapeDtypeStruct((NP, HID), f32),
        name="tc_node_tanh")(agg0, bias_0.reshape(1, HID))

    gather32 = pl.kernel(
        functools.partial(_gather_table_body, T),
        out_type=jax.ShapeDtypeStruct((EP, HID), f32),
        mesh=_sc_mesh(),
        scratch_types=[pltpu.VMEM((IR, 128), i32),
                       pltpu.VMEM((CH, HID), f32),
                       pltpu.SemaphoreType.DMA],
        compiler_params=_sc_params,
        name="sc_gather_feats")
    f1 = gather32(feats, srcp)

    msg1 = pl.pallas_call(
        _edge1_body,
        grid=(nb,),
        in_specs=[pl.BlockSpec((BR, 128), lambda i: (i, 0)),
                  pl.BlockSpec((BR, 128), lambda i: (i, 0)),
                  const_spec(128, 128)],
        out_specs=pl.BlockSpec((BR, 128), lambda i: (i, 0)),
        out_shape=jax.ShapeDtypeStruct((EP // 4, 128), f32),
        name="tc_edge1")(k1e, f1.reshape(EP // 4, 128), z4)

    agg1 = scatter32(msg1.reshape(EP, HID), dstp, zeros32)

    v0 = agg1[0, :, 0].reshape(NBLK, 128)
    v1 = agg1[1, :, 0].reshape(NBLK, 128)
    MOLP = 512
    molf, moli = pl.pallas_call(
        functools.partial(_pool_body, NBLK, MOLP),
        in_specs=[pl.BlockSpec((NBLK, 128), lambda: (0, 0)),
                  pl.BlockSpec((NBLK, 128), lambda: (0, 0)),
                  pl.BlockSpec((1, 1), lambda: (0, 0)),
                  pl.BlockSpec((NBLK, 128), lambda: (0, 0))],
        out_specs=[pl.BlockSpec((NUM_MOL, 1), lambda: (0, 0)),
                   pl.BlockSpec((NUM_MOL, 1), lambda: (0, 0))],
        out_shape=[jax.ShapeDtypeStruct((NUM_MOL, 1), f32),
                   jax.ShapeDtypeStruct((NUM_MOL, 1), jnp.int32)],
        name="tc_pool")(v0, v1, b1s, v2m)

    return (molf, molf, moli)


# speedup vs baseline: 5.5964x; 1.0151x over previous
import functools

import jax
import jax.numpy as jnp
from jax import lax
from jax.experimental import pallas as pl
from jax.experimental.pallas import tpu as pltpu
from jax.experimental.pallas import tpu_sc as plsc

RADIUS = 1.54
HID = 32
NUM_MOL = 500

NC, NS = 2, 16
NW = NC * NS
CH = 1024
IR = CH // 128

_sc_mesh = functools.partial(
    plsc.VectorSubcoreMesh, core_axis_name="c", subcore_axis_name="s",
    num_cores=NC, num_subcores=NS)
_sc_params = pltpu.CompilerParams(use_tc_tiling_on_sc=False)


def _worker_id():
    return lax.axis_index("s") * NC + lax.axis_index("c")


def _gather_edges_body(T, CH, IR, tsrc, srcp, dstp, a_out, d_out,
                       idx_s, idx_d, abuf, dbuf, sem, sem2):
    w = _worker_id()

    def chunk(t, carry):
        row0 = (w * T + t) * IR
        e0 = (w * T + t) * CH
        pltpu.sync_copy(srcp.at[pl.ds(row0, IR)], idx_s)
        pltpu.sync_copy(dstp.at[pl.ds(row0, IR)], idx_d)
        cps = [pltpu.async_copy(tsrc.at[idx_s.at[j]],
                                abuf.at[pl.ds(j * 128, 128)], sem)
               for j in range(IR)]
        cps += [pltpu.async_copy(tsrc.at[idx_d.at[j]],
                                 dbuf.at[pl.ds(j * 128, 128)], sem2)
                for j in range(IR)]
        for cp in cps:
            cp.wait()
        pltpu.sync_copy(abuf, a_out.at[pl.ds(e0, CH)])
        pltpu.sync_copy(dbuf, d_out.at[pl.ds(e0, CH)])
        return carry

    lax.fori_loop(0, T, chunk, 0)


def _gather_table_body(T, CH, IR, table, srcp, out, idx_s, buf, sem):
    w = _worker_id()

    def chunk(t, carry):
        row0 = (w * T + t) * IR
        e0 = (w * T + t) * CH
        pltpu.sync_copy(srcp.at[pl.ds(row0, IR)], idx_s)
        cps = [pltpu.async_copy(table.at[idx_s.at[j]],
                                buf.at[pl.ds(j * 128, 128)], sem)
               for j in range(IR)]
        for cp in cps:
            cp.wait()
        pltpu.sync_copy(buf, out.at[pl.ds(e0, CH)])
        return carry

    lax.fori_loop(0, T, chunk, 0)


def _scatter_add_body(T, NP, CH, IR, msg, dstp, zeros, out, idx_d, mbuf, table, sem):
    c = lax.axis_index("c")
    s = lax.axis_index("s")
    w = s * NC + c
    rpt = NP // NS
    pltpu.sync_copy(zeros.at[pl.ds(s * rpt, rpt)],
                    table.at[pl.ds(s * rpt, rpt)])
    plsc.subcore_barrier()

    def chunk(t, carry):
        row0 = (w * T + t) * IR
        e0 = (w * T + t) * CH
        pltpu.sync_copy(dstp.at[pl.ds(row0, IR)], idx_d)
        pltpu.sync_copy(msg.at[pl.ds(e0, CH)], mbuf)
        for j in range(IR):
            pltpu.sync_copy(mbuf.at[pl.ds(j * 128, 128)],
                            table.at[idx_d.at[j]], add=True)
        return carry

    lax.fori_loop(0, T, chunk, 0)
    plsc.subcore_barrier()
    pltpu.sync_copy(table.at[pl.ds(s * rpt, rpt)],
                    out.at[c].at[pl.ds(s * rpt, rpt)])


def _edge0_body(a_ref, d_ref, w10, b10r, w11, b11r, w21, b21r,
                w2p, r7, s224, b0m, msg_ref, k1_ref):
    a = a_ref[...]
    d = d_ref[...]
    for k in range(4):
        ak = a[:, k * HID:(k + 1) * HID]
        dk = d[:, k * HID:(k + 1) * HID]
        hoodk = ak[:, 7:9] - dk[:, 7:9]
        h0 = jnp.tanh(jnp.dot(hoodk, w10[...],
                              preferred_element_type=jnp.float32) + b10r[...])
        p = jnp.dot(h0, w2p[...], preferred_element_type=jnp.float32)
        fk = ak[:, 0:7]
        frep = jnp.dot(fk, r7[...], preferred_element_type=jnp.float32)
        acc = (jnp.dot(p * frep, s224[...],
                       preferred_element_type=jnp.float32)
               + jnp.dot(fk, b0m[...], preferred_element_type=jnp.float32))
        h1 = jnp.tanh(jnp.dot(hoodk, w11[...],
                              preferred_element_type=jnp.float32) + b11r[...])
        k1 = jnp.dot(h1, w21[...],
                     preferred_element_type=jnp.float32) + b21r[...]
        msg_ref[:, k * HID:(k + 1) * HID] = acc
        k1_ref[:, k * HID:(k + 1) * HID] = k1


def _node_tanh_body(p_ref, b_ref, f_ref):
    p = p_ref[...]
    f_ref[...] = jnp.tanh(p[0] + p[1] + b_ref[...])


def _edge1_body(k1_ref, f1_ref, z4, msg_ref):
    msg_ref[...] = jnp.dot(k1_ref[...] * f1_ref[...], z4[...],
                           preferred_element_type=jnp.float32)


def _pool_body(NB, MOLP, v0_ref, v1_ref, b1_ref, v2m_ref, molf_ref, moli_ref):
    bias = b1_ref[0, 0]
    mol_ids = lax.broadcasted_iota(jnp.int32, (MOLP, 128), 0)
    neg_inf = jnp.float32(-jnp.inf)
    int_min = jnp.int32(-2147483648)

    def pass1(k, cur):
        vals = v0_ref[pl.ds(k, 1), :] + v1_ref[pl.ds(k, 1), :] + bias
        seg = v2m_ref[pl.ds(k, 1), :]
        cand = jnp.where(seg == mol_ids, vals, neg_inf)
        return jnp.maximum(cur, jnp.max(cand, axis=1, keepdims=True))

    molmax = lax.fori_loop(0, NB, pass1,
                           jnp.full((MOLP, 1), neg_inf, jnp.float32))

    def pass2(k, cur):
        vals = v0_ref[pl.ds(k, 1), :] + v1_ref[pl.ds(k, 1), :] + bias
        seg = v2m_ref[pl.ds(k, 1), :]
        ids = lax.broadcasted_iota(jnp.int32, (MOLP, 128), 1) + k * 128
        mask = (seg == mol_ids) & (vals >= molmax)
        cand = jnp.where(mask, ids, int_min)
        return jnp.maximum(cur, jnp.max(cand, axis=1, keepdims=True))

    molidx = lax.fori_loop(0, NB, pass2,
                           jnp.full((MOLP, 1), int_min, jnp.int32))
    molf_ref[...] = molmax[0:NUM_MOL]
    moli_ref[...] = molidx[0:NUM_MOL]


def kernel(atom_features, atom_pos, molecule_edges, vertex2molecule,
           num_molecules, W1_0, b1_0, W2_0, b2_0, bias_0,
           W1_1, b1_1, W2_1, b2_1, bias_1):
    del num_molecules
    N, F = atom_features.shape
    E = molecule_edges.shape[0]
    f32, i32 = jnp.float32, jnp.int32

    NP = ((N + 255) // 256) * 256
    EPW = NW * CH
    EP = ((E + EPW - 1) // EPW) * EPW
    T = EP // (NW * CH)
    CH2, IR2 = 2048, 16
    BE = 8192
    NBLK = NP // 128

    src = molecule_edges[:, 0]
    dst = molecule_edges[:, 1]
    padi = jnp.full((EP - E,), N, dtype=i32)
    srcp = jnp.concatenate([src, padi]).reshape(EP // 128, 128)
    dstp = jnp.concatenate([dst, padi]).reshape(EP // 128, 128)

    tsrc = jnp.pad(jnp.concatenate([atom_features, atom_pos], axis=1),
                   ((0, NP - N), (0, HID - F - 2)))

    zeros32 = jnp.zeros((NP, HID), f32)

    inv_r = 1.0 / RADIUS
    w10 = W1_0 * inv_r
    w11 = W1_1 * inv_r
    b10r = b1_0.reshape(1, HID)
    b11r = b1_1.reshape(1, HID)
    b21r = b2_1.reshape(1, HID)
    w21 = W2_1
    w2p = W2_0.reshape(HID, HID, F).transpose(0, 2, 1).reshape(HID, HID * F)
    r7 = jnp.kron(jnp.eye(F, dtype=f32), jnp.ones((1, HID), f32))
    s224 = jnp.tile(jnp.eye(HID, dtype=f32), (F, 1))
    b0m = b2_0.reshape(HID, F).T
    zpat = jnp.zeros((HID, HID), f32).at[:, 0].set(1.0)
    z4 = jnp.kron(jnp.eye(4, dtype=f32), zpat)
    b1s = bias_1.reshape(1, 1)
    v2m = jnp.concatenate(
        [vertex2molecule, jnp.full((NP - N,), -1, i32)]).reshape(NBLK, 128)

    gather_edges = pl.kernel(
        functools.partial(_gather_edges_body, T, CH, IR),
        out_type=[jax.ShapeDtypeStruct((EP, HID), f32),
                  jax.ShapeDtypeStruct((EP, HID), f32)],
        mesh=_sc_mesh(),
        scratch_types=[pltpu.VMEM((IR, 128), i32),
                       pltpu.VMEM((IR, 128), i32),
                       pltpu.VMEM((CH, HID), f32),
                       pltpu.VMEM((CH, HID), f32),
                       pltpu.SemaphoreType.DMA,
                       pltpu.SemaphoreType.DMA],
        compiler_params=_sc_params,
        name="sc_gather_edges")
    a_e, d_e = gather_edges(tsrc, srcp, dstp)

    nb = EP // BE
    BR = BE // 4
    a4 = a_e.reshape(EP // 4, 128)
    d4 = d_e.reshape(EP // 4, 128)
    const_spec = lambda r, c: pl.BlockSpec((r, c), lambda i: (0, 0))
    msg0, k1e = pl.pallas_call(
        _edge0_body,
        grid=(nb,),
        in_specs=[pl.BlockSpec((BR, 128), lambda i: (i, 0)),
                  pl.BlockSpec((BR, 128), lambda i: (i, 0)),
                  const_spec(2, HID), const_spec(1, HID),
                  const_spec(2, HID), const_spec(1, HID),
                  const_spec(HID, HID), const_spec(1, HID),
                  const_spec(HID, HID * F), const_spec(F, HID * F),
                  const_spec(HID * F, HID), const_spec(F, HID)],
        out_specs=[pl.BlockSpec((BR, 128), lambda i: (i, 0)),
                   pl.BlockSpec((BR, 128), lambda i: (i, 0))],
        out_shape=[jax.ShapeDtypeStruct((EP // 4, 128), f32),
                   jax.ShapeDtypeStruct((EP // 4, 128), f32)],
        name="tc_edge0")(a4, d4, w10, b10r, w11, b11r, w21, b21r,
                         w2p, r7, s224, b0m)

    T2 = EP // (NW * CH2)
    scatter32 = pl.kernel(
        functools.partial(_scatter_add_body, T2, NP, CH2, IR2),
        out_type=jax.ShapeDtypeStruct((NC, NP, HID), f32),
        mesh=_sc_mesh(),
        scratch_types=[pltpu.VMEM((IR2, 128), i32),
                       pltpu.VMEM((CH2, HID), f32),
                       pltpu.VMEM_SHARED((NP, HID), f32),
                       pltpu.SemaphoreType.DMA],
        compiler_params=_sc_params,
        name="sc_scatter_msg0")
    agg0 = scatter32(msg0.reshape(EP, HID), dstp, zeros32)

    feats = pl.pallas_call(
        _node_tanh_body,
        in_specs=[pl.BlockSpec((NC, NP, HID), lambda: (0, 0, 0)),
                  pl.BlockSpec((1, HID), lambda: (0, 0))],
        out_specs=pl.BlockSpec((NP, HID), lambda: (0, 0)),
        out_shape=jax.ShapeDtypeStruct((NP, HID), f32),
        name="tc_node_tanh")(agg0, bias_0.reshape(1, HID))

    gather32 = pl.kernel(
        functools.partial(_gather_table_body, T2, CH2, IR2),
        out_type=jax.ShapeDtypeStruct((EP, HID), f32),
        mesh=_sc_mesh(),
        scratch_types=[pltpu.VMEM((IR2, 128), i32),
                       pltpu.VMEM((CH2, HID), f32),
                       pltpu.SemaphoreType.DMA],
        compiler_params=_sc_params,
        name="sc_gather_feats")
    f1 = gather32(feats, srcp)

    msg1 = pl.pallas_call(
        _edge1_body,
        grid=(nb,),
        in_specs=[pl.BlockSpec((BR, 128), lambda i: (i, 0)),
                  pl.BlockSpec((BR, 128), lambda i: (i, 0)),
                  const_spec(128, 128)],
        out_specs=pl.BlockSpec((BR, 128), lambda i: (i, 0)),
        out_shape=jax.ShapeDtypeStruct((EP // 4, 128), f32),
        name="tc_edge1")(k1e, f1.reshape(EP // 4, 128), z4)

    agg1 = scatter32(msg1.reshape(EP, HID), dstp, zeros32)

    v0 = agg1[0, :, 0].reshape(NBLK, 128)
    v1 = agg1[1, :, 0].reshape(NBLK, 128)
    MOLP = 512
    molf, moli = pl.pallas_call(
        functools.partial(_pool_body, NBLK, MOLP),
        in_specs=[pl.BlockSpec((NBLK, 128), lambda: (0, 0)),
                  pl.BlockSpec((NBLK, 128), lambda: (0, 0)),
                  pl.BlockSpec((1, 1), lambda: (0, 0)),
                  pl.BlockSpec((NBLK, 128), lambda: (0, 0))],
        out_specs=[pl.BlockSpec((NUM_MOL, 1), lambda: (0, 0)),
                   pl.BlockSpec((NUM_MOL, 1), lambda: (0, 0))],
        out_shape=[jax.ShapeDtypeStruct((NUM_MOL, 1), f32),
                   jax.ShapeDtypeStruct((NUM_MOL, 1), jnp.int32)],
        name="tc_pool")(v0, v1, b1s, v2m)

    return (molf, molf, moli)


# speedup vs baseline: 8.6245x; 1.5411x over previous
import functools

import jax
import jax.numpy as jnp
from jax import lax
from jax.experimental import pallas as pl
from jax.experimental.pallas import tpu as pltpu
from jax.experimental.pallas import tpu_sc as plsc

RADIUS = 1.54
HID = 32
NUM_MOL = 500

NC, NS = 2, 16
NW = NC * NS
CH = 1024
IR = CH // 128

_sc_mesh = functools.partial(
    plsc.VectorSubcoreMesh, core_axis_name="c", subcore_axis_name="s",
    num_cores=NC, num_subcores=NS)
_sc_params = pltpu.CompilerParams(use_tc_tiling_on_sc=False)


def _worker_id():
    return lax.axis_index("s") * NC + lax.axis_index("c")


def _gather_edges_body(T, CH, IR, NP, tsrc, srcp, dstp, a_out, d_out,
                       idx_s, idx_d, abuf, dbuf, tspm, sem, sem2):
    w = _worker_id()
    s = lax.axis_index("s")
    rpt = NP // NS
    pltpu.sync_copy(tsrc.at[pl.ds(s * rpt, rpt)],
                    tspm.at[pl.ds(s * rpt, rpt)])
    plsc.subcore_barrier()

    def chunk(t, carry):
        row0 = (w * T + t) * IR
        e0 = (w * T + t) * CH
        pltpu.sync_copy(srcp.at[pl.ds(row0, IR)], idx_s)
        pltpu.sync_copy(dstp.at[pl.ds(row0, IR)], idx_d)
        cps = [pltpu.async_copy(tspm.at[idx_s.at[j]],
                                abuf.at[pl.ds(j * 128, 128)], sem)
               for j in range(IR)]
        cps += [pltpu.async_copy(tspm.at[idx_d.at[j]],
                                 dbuf.at[pl.ds(j * 128, 128)], sem2)
                for j in range(IR)]
        for cp in cps:
            cp.wait()
        pltpu.sync_copy(abuf, a_out.at[pl.ds(e0, CH)])
        pltpu.sync_copy(dbuf, d_out.at[pl.ds(e0, CH)])
        return carry

    lax.fori_loop(0, T, chunk, 0)


def _gather_table_body(T, CH, IR, NP, table, srcp, out, idx_s, buf, tspm, sem):
    w = _worker_id()
    s = lax.axis_index("s")
    rpt = NP // NS
    pltpu.sync_copy(table.at[pl.ds(s * rpt, rpt)],
                    tspm.at[pl.ds(s * rpt, rpt)])
    plsc.subcore_barrier()

    def chunk(t, carry):
        row0 = (w * T + t) * IR
        e0 = (w * T + t) * CH
        pltpu.sync_copy(srcp.at[pl.ds(row0, IR)], idx_s)
        cps = [pltpu.async_copy(tspm.at[idx_s.at[j]],
                                buf.at[pl.ds(j * 128, 128)], sem)
               for j in range(IR)]
        for cp in cps:
            cp.wait()
        pltpu.sync_copy(buf, out.at[pl.ds(e0, CH)])
        return carry

    lax.fori_loop(0, T, chunk, 0)


def _scatter_add_body(T, NP, CH, IR, msg, dstp, zeros, out, idx_d, mbuf, table, sem):
    c = lax.axis_index("c")
    s = lax.axis_index("s")
    w = s * NC + c
    rpt = NP // NS
    pltpu.sync_copy(zeros.at[pl.ds(s * rpt, rpt)],
                    table.at[pl.ds(s * rpt, rpt)])
    plsc.subcore_barrier()

    def chunk(t, carry):
        row0 = (w * T + t) * IR
        e0 = (w * T + t) * CH
        pltpu.sync_copy(dstp.at[pl.ds(row0, IR)], idx_d)
        pltpu.sync_copy(msg.at[pl.ds(e0, CH)], mbuf)
        for j in range(IR):
            pltpu.sync_copy(mbuf.at[pl.ds(j * 128, 128)],
                            table.at[idx_d.at[j]], add=True)
        return carry

    lax.fori_loop(0, T, chunk, 0)
    plsc.subcore_barrier()
    pltpu.sync_copy(table.at[pl.ds(s * rpt, rpt)],
                    out.at[c].at[pl.ds(s * rpt, rpt)])


def _edge0_body(a_ref, d_ref, w10, b10r, w11, b11r, w21, b21r,
                w2p, r7, s224, b0m, msg_ref, k1_ref):
    a = a_ref[...]
    d = d_ref[...]
    for k in range(4):
        ak = a[:, k * HID:(k + 1) * HID]
        dk = d[:, k * HID:(k + 1) * HID]
        hoodk = ak[:, 7:9] - dk[:, 7:9]
        h0 = jnp.tanh(jnp.dot(hoodk, w10[...],
                              preferred_element_type=jnp.float32) + b10r[...])
        p = jnp.dot(h0, w2p[...], preferred_element_type=jnp.float32)
        fk = ak[:, 0:7]
        frep = jnp.dot(fk, r7[...], preferred_element_type=jnp.float32)
        acc = (jnp.dot(p * frep, s224[...],
                       preferred_element_type=jnp.float32)
               + jnp.dot(fk, b0m[...], preferred_element_type=jnp.float32))
        h1 = jnp.tanh(jnp.dot(hoodk, w11[...],
                              preferred_element_type=jnp.float32) + b11r[...])
        k1 = jnp.dot(h1, w21[...],
                     preferred_element_type=jnp.float32) + b21r[...]
        msg_ref[:, k * HID:(k + 1) * HID] = acc
        k1_ref[:, k * HID:(k + 1) * HID] = k1


def _node_tanh_body(p_ref, b_ref, f_ref):
    p = p_ref[...]
    f_ref[...] = jnp.tanh(p[0] + p[1] + b_ref[...])


def _edge1_body(k1_ref, f1_ref, z4, msg_ref):
    msg_ref[...] = jnp.dot(k1_ref[...] * f1_ref[...], z4[...],
                           preferred_element_type=jnp.float32)


def _pool_body(NB, MOLP, v0_ref, v1_ref, b1_ref, v2m_ref, molf_ref, moli_ref):
    bias = b1_ref[0, 0]
    mol_ids = lax.broadcasted_iota(jnp.int32, (MOLP, 128), 0)
    neg_inf = jnp.float32(-jnp.inf)
    int_min = jnp.int32(-2147483648)

    def pass1(k, cur):
        vals = v0_ref[pl.ds(k, 1), :] + v1_ref[pl.ds(k, 1), :] + bias
        seg = v2m_ref[pl.ds(k, 1), :]
        cand = jnp.where(seg == mol_ids, vals, neg_inf)
        return jnp.maximum(cur, jnp.max(cand, axis=1, keepdims=True))

    molmax = lax.fori_loop(0, NB, pass1,
                           jnp.full((MOLP, 1), neg_inf, jnp.float32))

    def pass2(k, cur):
        vals = v0_ref[pl.ds(k, 1), :] + v1_ref[pl.ds(k, 1), :] + bias
        seg = v2m_ref[pl.ds(k, 1), :]
        ids = lax.broadcasted_iota(jnp.int32, (MOLP, 128), 1) + k * 128
        mask = (seg == mol_ids) & (vals >= molmax)
        cand = jnp.where(mask, ids, int_min)
        return jnp.maximum(cur, jnp.max(cand, axis=1, keepdims=True))

    molidx = lax.fori_loop(0, NB, pass2,
                           jnp.full((MOLP, 1), int_min, jnp.int32))
    molf_ref[...] = molmax[0:NUM_MOL]
    moli_ref[...] = molidx[0:NUM_MOL]


def kernel(atom_features, atom_pos, molecule_edges, vertex2molecule,
           num_molecules, W1_0, b1_0, W2_0, b2_0, bias_0,
           W1_1, b1_1, W2_1, b2_1, bias_1):
    del num_molecules
    N, F = atom_features.shape
    E = molecule_edges.shape[0]
    f32, i32 = jnp.float32, jnp.int32

    NP = ((N + 255) // 256) * 256
    EPW = NW * CH
    EP = ((E + EPW - 1) // EPW) * EPW
    T = EP // (NW * CH)
    CH2, IR2 = 2048, 16
    BE = 8192
    NBLK = NP // 128

    src = molecule_edges[:, 0]
    dst = molecule_edges[:, 1]
    padi = jnp.full((EP - E,), N, dtype=i32)
    srcp = jnp.concatenate([src, padi]).reshape(EP // 128, 128)
    dstp = jnp.concatenate([dst, padi]).reshape(EP // 128, 128)

    tsrc = jnp.pad(jnp.concatenate([atom_features, atom_pos], axis=1),
                   ((0, NP - N), (0, HID - F - 2)))

    zeros32 = jnp.zeros((NP, HID), f32)

    inv_r = 1.0 / RADIUS
    w10 = W1_0 * inv_r
    w11 = W1_1 * inv_r
    b10r = b1_0.reshape(1, HID)
    b11r = b1_1.reshape(1, HID)
    b21r = b2_1.reshape(1, HID)
    w21 = W2_1
    w2p = W2_0.reshape(HID, HID, F).transpose(0, 2, 1).reshape(HID, HID * F)
    r7 = jnp.kron(jnp.eye(F, dtype=f32), jnp.ones((1, HID), f32))
    s224 = jnp.tile(jnp.eye(HID, dtype=f32), (F, 1))
    b0m = b2_0.reshape(HID, F).T
    zpat = jnp.zeros((HID, HID), f32).at[:, 0].set(1.0)
    z4 = jnp.kron(jnp.eye(4, dtype=f32), zpat)
    b1s = bias_1.reshape(1, 1)
    v2m = jnp.concatenate(
        [vertex2molecule, jnp.full((NP - N,), -1, i32)]).reshape(NBLK, 128)

    gather_edges = pl.kernel(
        functools.partial(_gather_edges_body, T, CH, IR, NP),
        out_type=[jax.ShapeDtypeStruct((EP, HID), f32),
                  jax.ShapeDtypeStruct((EP, HID), f32)],
        mesh=_sc_mesh(),
        scratch_types=[pltpu.VMEM((IR, 128), i32),
                       pltpu.VMEM((IR, 128), i32),
                       pltpu.VMEM((CH, HID), f32),
                       pltpu.VMEM((CH, HID), f32),
                       pltpu.VMEM_SHARED((NP, HID), f32),
                       pltpu.SemaphoreType.DMA,
                       pltpu.SemaphoreType.DMA],
        compiler_params=_sc_params,
        name="sc_gather_edges")
    a_e, d_e = gather_edges(tsrc, srcp, dstp)

    nb = EP // BE
    BR = BE // 4
    a4 = a_e.reshape(EP // 4, 128)
    d4 = d_e.reshape(EP // 4, 128)
    const_spec = lambda r, c: pl.BlockSpec((r, c), lambda i: (0, 0))
    msg0, k1e = pl.pallas_call(
        _edge0_body,
        grid=(nb,),
        in_specs=[pl.BlockSpec((BR, 128), lambda i: (i, 0)),
                  pl.BlockSpec((BR, 128), lambda i: (i, 0)),
                  const_spec(2, HID), const_spec(1, HID),
                  const_spec(2, HID), const_spec(1, HID),
                  const_spec(HID, HID), const_spec(1, HID),
                  const_spec(HID, HID * F), const_spec(F, HID * F),
                  const_spec(HID * F, HID), const_spec(F, HID)],
        out_specs=[pl.BlockSpec((BR, 128), lambda i: (i, 0)),
                   pl.BlockSpec((BR, 128), lambda i: (i, 0))],
        out_shape=[jax.ShapeDtypeStruct((EP // 4, 128), f32),
                   jax.ShapeDtypeStruct((EP // 4, 128), f32)],
        name="tc_edge0")(a4, d4, w10, b10r, w11, b11r, w21, b21r,
                         w2p, r7, s224, b0m)

    T2 = EP // (NW * CH2)
    scatter32 = pl.kernel(
        functools.partial(_scatter_add_body, T2, NP, CH2, IR2),
        out_type=jax.ShapeDtypeStruct((NC, NP, HID), f32),
        mesh=_sc_mesh(),
        scratch_types=[pltpu.VMEM((IR2, 128), i32),
                       pltpu.VMEM((CH2, HID), f32),
                       pltpu.VMEM_SHARED((NP, HID), f32),
                       pltpu.SemaphoreType.DMA],
        compiler_params=_sc_params,
        name="sc_scatter_msg0")
    agg0 = scatter32(msg0.reshape(EP, HID), dstp, zeros32)

    feats = pl.pallas_call(
        _node_tanh_body,
        in_specs=[pl.BlockSpec((NC, NP, HID), lambda: (0, 0, 0)),
                  pl.BlockSpec((1, HID), lambda: (0, 0))],
        out_specs=pl.BlockSpec((NP, HID), lambda: (0, 0)),
        out_shape=jax.ShapeDtypeStruct((NP, HID), f32),
        name="tc_node_tanh")(agg0, bias_0.reshape(1, HID))

    gather32 = pl.kernel(
        functools.partial(_gather_table_body, T2, CH2, IR2, NP),
        out_type=jax.ShapeDtypeStruct((EP, HID), f32),
        mesh=_sc_mesh(),
        scratch_types=[pltpu.VMEM((IR2, 128), i32),
                       pltpu.VMEM((CH2, HID), f32),
                       pltpu.VMEM_SHARED((NP, HID), f32),
                       pltpu.SemaphoreType.DMA],
        compiler_params=_sc_params,
        name="sc_gather_feats")
    f1 = gather32(feats, srcp)

    msg1 = pl.pallas_call(
        _edge1_body,
        grid=(nb,),
        in_specs=[pl.BlockSpec((BR, 128), lambda i: (i, 0)),
                  pl.BlockSpec((BR, 128), lambda i: (i, 0)),
                  const_spec(128, 128)],
        out_specs=pl.BlockSpec((BR, 128), lambda i: (i, 0)),
        out_shape=jax.ShapeDtypeStruct((EP // 4, 128), f32),
        name="tc_edge1")(k1e, f1.reshape(EP // 4, 128), z4)

    agg1 = scatter32(msg1.reshape(EP, HID), dstp, zeros32)

    v0 = agg1[0, :, 0].reshape(NBLK, 128)
    v1 = agg1[1, :, 0].reshape(NBLK, 128)
    MOLP = 512
    molf, moli = pl.pallas_call(
        functools.partial(_pool_body, NBLK, MOLP),
        in_specs=[pl.BlockSpec((NBLK, 128), lambda: (0, 0)),
                  pl.BlockSpec((NBLK, 128), lambda: (0, 0)),
                  pl.BlockSpec((1, 1), lambda: (0, 0)),
                  pl.BlockSpec((NBLK, 128), lambda: (0, 0))],
        out_specs=[pl.BlockSpec((NUM_MOL, 1), lambda: (0, 0)),
                   pl.BlockSpec((NUM_MOL, 1), lambda: (0, 0))],
        out_shape=[jax.ShapeDtypeStruct((NUM_MOL, 1), f32),
                   jax.ShapeDtypeStruct((NUM_MOL, 1), jnp.int32)],
        name="tc_pool")(v0, v1, b1s, v2m)

    return (molf, molf, moli)


# speedup vs baseline: 8.8802x; 1.0296x over previous
import functools

import jax
import jax.numpy as jnp
from jax import lax
from jax.experimental import pallas as pl
from jax.experimental.pallas import tpu as pltpu
from jax.experimental.pallas import tpu_sc as plsc

RADIUS = 1.54
HID = 32
NUM_MOL = 500

NC, NS = 2, 16
NW = NC * NS
CH = 1024
IR = CH // 128

_sc_mesh = functools.partial(
    plsc.VectorSubcoreMesh, core_axis_name="c", subcore_axis_name="s",
    num_cores=NC, num_subcores=NS)
_sc_params = pltpu.CompilerParams(use_tc_tiling_on_sc=False)


def _worker_id():
    return lax.axis_index("s") * NC + lax.axis_index("c")


def _gather_edges_body(T, CH, IR, NP, tsrc, srcp, dstp, a_out, d_out,
                       idx_s, idx_d, abuf, dbuf, tspm, sem, sem2):
    w = _worker_id()
    s = lax.axis_index("s")
    rpt = NP // NS
    pltpu.sync_copy(tsrc.at[pl.ds(s * rpt, rpt)],
                    tspm.at[pl.ds(s * rpt, rpt)])
    plsc.subcore_barrier()

    def chunk(t, carry):
        row0 = (w * T + t) * IR
        e0 = (w * T + t) * CH
        pltpu.sync_copy(srcp.at[pl.ds(row0, IR)], idx_s)
        pltpu.sync_copy(dstp.at[pl.ds(row0, IR)], idx_d)
        cps = [pltpu.async_copy(tspm.at[idx_s.at[j]],
                                abuf.at[pl.ds(j * 128, 128)], sem)
               for j in range(IR)]
        cps += [pltpu.async_copy(tspm.at[idx_d.at[j]],
                                 dbuf.at[pl.ds(j * 128, 128)], sem2)
                for j in range(IR)]
        for cp in cps:
            cp.wait()
        pltpu.sync_copy(abuf, a_out.at[pl.ds(e0, CH)])
        pltpu.sync_copy(dbuf, d_out.at[pl.ds(e0, CH)])
        return carry

    lax.fori_loop(0, T, chunk, 0)


def _gather_table_body(T, CH, IR, NP, table, srcp, out, idx_s, buf, tspm, sem):
    w = _worker_id()
    s = lax.axis_index("s")
    rpt = NP // NS
    pltpu.sync_copy(table.at[pl.ds(s * rpt, rpt)],
                    tspm.at[pl.ds(s * rpt, rpt)])
    plsc.subcore_barrier()

    def chunk(t, carry):
        row0 = (w * T + t) * IR
        e0 = (w * T + t) * CH
        pltpu.sync_copy(srcp.at[pl.ds(row0, IR)], idx_s)
        cps = [pltpu.async_copy(tspm.at[idx_s.at[j]],
                                buf.at[pl.ds(j * 128, 128)], sem)
               for j in range(IR)]
        for cp in cps:
            cp.wait()
        pltpu.sync_copy(buf, out.at[pl.ds(e0, CH)])
        return carry

    lax.fori_loop(0, T, chunk, 0)


def _scatter_add_body(T, NP, CH, IR, msg, dstp, zeros, out, idx_d, mbuf, table, sem):
    c = lax.axis_index("c")
    s = lax.axis_index("s")
    w = s * NC + c
    rpt = NP // NS
    pltpu.sync_copy(zeros.at[pl.ds(s * rpt, rpt)],
                    table.at[pl.ds(s * rpt, rpt)])
    plsc.subcore_barrier()

    def chunk(t, carry):
        row0 = (w * T + t) * IR
        e0 = (w * T + t) * CH
        pltpu.sync_copy(dstp.at[pl.ds(row0, IR)], idx_d)
        pltpu.sync_copy(msg.at[pl.ds(e0, CH)], mbuf)
        for j in range(IR):
            pltpu.sync_copy(mbuf.at[pl.ds(j * 128, 128)],
                            table.at[idx_d.at[j]], add=True)
        return carry

    lax.fori_loop(0, T, chunk, 0)
    plsc.subcore_barrier()
    pltpu.sync_copy(table.at[pl.ds(s * rpt, rpt)],
                    out.at[c].at[pl.ds(s * rpt, rpt)])


def _edge0_body(a_ref, d_ref, w10, b10r, w11, b11r, w21, b21r,
                w2p, r7, s224, b0m, msg_ref, k1_ref):
    a = a_ref[...]
    d = d_ref[...]
    for k in range(4):
        ak = a[:, k * HID:(k + 1) * HID]
        dk = d[:, k * HID:(k + 1) * HID]
        hoodk = ak[:, 7:9] - dk[:, 7:9]
        h0 = jnp.tanh(jnp.dot(hoodk, w10[...],
                              preferred_element_type=jnp.float32) + b10r[...])
        p = jnp.dot(h0, w2p[...], preferred_element_type=jnp.float32)
        fk = ak[:, 0:7]
        frep = jnp.dot(fk, r7[...], preferred_element_type=jnp.float32)
        acc = (jnp.dot(p * frep, s224[...],
                       preferred_element_type=jnp.float32)
               + jnp.dot(fk, b0m[...], preferred_element_type=jnp.float32))
        h1 = jnp.tanh(jnp.dot(hoodk, w11[...],
                              preferred_element_type=jnp.float32) + b11r[...])
        k1 = jnp.dot(h1, w21[...],
                     preferred_element_type=jnp.float32) + b21r[...]
        msg_ref[:, k * HID:(k + 1) * HID] = acc
        k1_ref[:, k * HID:(k + 1) * HID] = k1


def _node_tanh_body(p_ref, b_ref, f_ref):
    p = p_ref[...]
    f_ref[...] = jnp.tanh(p[0] + p[1] + b_ref[...])


def _edge1_body(k1_ref, f1_ref, z4, msg_ref):
    msg_ref[...] = jnp.dot(k1_ref[...] * f1_ref[...], z4[...],
                           preferred_element_type=jnp.float32)


def _pool_body(NB, MOLP, v0_ref, v1_ref, b1_ref, v2m_ref, molf_ref, moli_ref):
    bias = b1_ref[0, 0]
    mol_ids = lax.broadcasted_iota(jnp.int32, (MOLP, 128), 0)
    neg_inf = jnp.float32(-jnp.inf)
    int_min = jnp.int32(-2147483648)

    UNR = 8

    def pass1(k, cur):
        for j in range(UNR):
            r = k * UNR + j
            vals = v0_ref[pl.ds(r, 1), :] + v1_ref[pl.ds(r, 1), :] + bias
            seg = v2m_ref[pl.ds(r, 1), :]
            cand = jnp.where(seg == mol_ids, vals, neg_inf)
            cur = jnp.maximum(cur, jnp.max(cand, axis=1, keepdims=True))
        return cur

    molmax = lax.fori_loop(0, NB // UNR, pass1,
                           jnp.full((MOLP, 1), neg_inf, jnp.float32))

    def pass2(k, cur):
        for j in range(UNR):
            r = k * UNR + j
            vals = v0_ref[pl.ds(r, 1), :] + v1_ref[pl.ds(r, 1), :] + bias
            seg = v2m_ref[pl.ds(r, 1), :]
            ids = lax.broadcasted_iota(jnp.int32, (MOLP, 128), 1) + r * 128
            mask = (seg == mol_ids) & (vals >= molmax)
            cand = jnp.where(mask, ids, int_min)
            cur = jnp.maximum(cur, jnp.max(cand, axis=1, keepdims=True))
        return cur

    molidx = lax.fori_loop(0, NB // UNR, pass2,
                           jnp.full((MOLP, 1), int_min, jnp.int32))
    molf_ref[...] = molmax[0:NUM_MOL]
    moli_ref[...] = molidx[0:NUM_MOL]


def kernel(atom_features, atom_pos, molecule_edges, vertex2molecule,
           num_molecules, W1_0, b1_0, W2_0, b2_0, bias_0,
           W1_1, b1_1, W2_1, b2_1, bias_1):
    del num_molecules
    N, F = atom_features.shape
    E = molecule_edges.shape[0]
    f32, i32 = jnp.float32, jnp.int32

    NP = ((N + 255) // 256) * 256
    EPW = NW * CH
    EP = ((E + EPW - 1) // EPW) * EPW
    T = EP // (NW * CH)
    CH2, IR2 = 2048, 16
    BE = 8192
    NBLK = NP // 128

    src = molecule_edges[:, 0]
    dst = molecule_edges[:, 1]
    padi = jnp.full((EP - E,), N, dtype=i32)
    srcp = jnp.concatenate([src, padi]).reshape(EP // 128, 128)
    dstp = jnp.concatenate([dst, padi]).reshape(EP // 128, 128)

    tsrc = jnp.pad(jnp.concatenate([atom_features, atom_pos], axis=1),
                   ((0, NP - N), (0, HID - F - 2)))

    zeros32 = jnp.zeros((NP, HID), f32)

    inv_r = 1.0 / RADIUS
    w10 = W1_0 * inv_r
    w11 = W1_1 * inv_r
    b10r = b1_0.reshape(1, HID)
    b11r = b1_1.reshape(1, HID)
    b21r = b2_1.reshape(1, HID)
    w21 = W2_1
    w2p = W2_0.reshape(HID, HID, F).transpose(0, 2, 1).reshape(HID, HID * F)
    r7 = jnp.kron(jnp.eye(F, dtype=f32), jnp.ones((1, HID), f32))
    s224 = jnp.tile(jnp.eye(HID, dtype=f32), (F, 1))
    b0m = b2_0.reshape(HID, F).T
    zpat = jnp.zeros((HID, HID), f32).at[:, 0].set(1.0)
    z4 = jnp.kron(jnp.eye(4, dtype=f32), zpat)
    b1s = bias_1.reshape(1, 1)
    v2m = jnp.concatenate(
        [vertex2molecule, jnp.full((NP - N,), -1, i32)]).reshape(NBLK, 128)

    gather_edges = pl.kernel(
        functools.partial(_gather_edges_body, T, CH, IR, NP),
        out_type=[jax.ShapeDtypeStruct((EP, HID), f32),
                  jax.ShapeDtypeStruct((EP, HID), f32)],
        mesh=_sc_mesh(),
        scratch_types=[pltpu.VMEM((IR, 128), i32),
                       pltpu.VMEM((IR, 128), i32),
                       pltpu.VMEM((CH, HID), f32),
                       pltpu.VMEM((CH, HID), f32),
                       pltpu.VMEM_SHARED((NP, HID), f32),
                       pltpu.SemaphoreType.DMA,
                       pltpu.SemaphoreType.DMA],
        compiler_params=_sc_params,
        name="sc_gather_edges")
    a_e, d_e = gather_edges(tsrc, srcp, dstp)

    nb = EP // BE
    BR = BE // 4
    a4 = a_e.reshape(EP // 4, 128)
    d4 = d_e.reshape(EP // 4, 128)
    const_spec = lambda r, c: pl.BlockSpec((r, c), lambda i: (0, 0))
    msg0, k1e = pl.pallas_call(
        _edge0_body,
        grid=(nb,),
        in_specs=[pl.BlockSpec((BR, 128), lambda i: (i, 0)),
                  pl.BlockSpec((BR, 128), lambda i: (i, 0)),
                  const_spec(2, HID), const_spec(1, HID),
                  const_spec(2, HID), const_spec(1, HID),
                  const_spec(HID, HID), const_spec(1, HID),
                  const_spec(HID, HID * F), const_spec(F, HID * F),
                  const_spec(HID * F, HID), const_spec(F, HID)],
        out_specs=[pl.BlockSpec((BR, 128), lambda i: (i, 0)),
                   pl.BlockSpec((BR, 128), lambda i: (i, 0))],
        out_shape=[jax.ShapeDtypeStruct((EP // 4, 128), f32),
                   jax.ShapeDtypeStruct((EP // 4, 128), f32)],
        name="tc_edge0")(a4, d4, w10, b10r, w11, b11r, w21, b21r,
                         w2p, r7, s224, b0m)

    T2 = EP // (NW * CH2)
    scatter32 = pl.kernel(
        functools.partial(_scatter_add_body, T2, NP, CH2, IR2),
        out_type=jax.ShapeDtypeStruct((NC, NP, HID), f32),
        mesh=_sc_mesh(),
        scratch_types=[pltpu.VMEM((IR2, 128), i32),
                       pltpu.VMEM((CH2, HID), f32),
                       pltpu.VMEM_SHARED((NP, HID), f32),
                       pltpu.SemaphoreType.DMA],
        compiler_params=_sc_params,
        name="sc_scatter_msg0")
    agg0 = scatter32(msg0.reshape(EP, HID), dstp, zeros32)

    feats = pl.pallas_call(
        _node_tanh_body,
        in_specs=[pl.BlockSpec((NC, NP, HID), lambda: (0, 0, 0)),
                  pl.BlockSpec((1, HID), lambda: (0, 0))],
        out_specs=pl.BlockSpec((NP, HID), lambda: (0, 0)),
        out_shape=jax.ShapeDtypeStruct((NP, HID), f32),
        name="tc_node_tanh")(agg0, bias_0.reshape(1, HID))

    gather32 = pl.kernel(
        functools.partial(_gather_table_body, T2, CH2, IR2, NP),
        out_type=jax.ShapeDtypeStruct((EP, HID), f32),
        mesh=_sc_mesh(),
        scratch_types=[pltpu.VMEM((IR2, 128), i32),
                       pltpu.VMEM((CH2, HID), f32),
                       pltpu.VMEM_SHARED((NP, HID), f32),
                       pltpu.SemaphoreType.DMA],
        compiler_params=_sc_params,
        name="sc_gather_feats")
    f1 = gather32(feats, srcp)

    msg1 = pl.pallas_call(
        _edge1_body,
        grid=(nb,),
        in_specs=[pl.BlockSpec((BR, 128), lambda i: (i, 0)),
                  pl.BlockSpec((BR, 128), lambda i: (i, 0)),
                  const_spec(128, 128)],
        out_specs=pl.BlockSpec((BR, 128), lambda i: (i, 0)),
        out_shape=jax.ShapeDtypeStruct((EP // 4, 128), f32),
        name="tc_edge1")(k1e, f1.reshape(EP // 4, 128), z4)

    agg1 = scatter32(msg1.reshape(EP, HID), dstp, zeros32)

    v0 = agg1[0, :, 0].reshape(NBLK, 128)
    v1 = agg1[1, :, 0].reshape(NBLK, 128)
    MOLP = 512
    molf, moli = pl.pallas_call(
        functools.partial(_pool_body, NBLK, MOLP),
        in_specs=[pl.BlockSpec((NBLK, 128), lambda: (0, 0)),
                  pl.BlockSpec((NBLK, 128), lambda: (0, 0)),
                  pl.BlockSpec((1, 1), lambda: (0, 0)),
                  pl.BlockSpec((NBLK, 128), lambda: (0, 0))],
        out_specs=[pl.BlockSpec((NUM_MOL, 1), lambda: (0, 0)),
                   pl.BlockSpec((NUM_MOL, 1), lambda: (0, 0))],
        out_shape=[jax.ShapeDtypeStruct((NUM_MOL, 1), f32),
                   jax.ShapeDtypeStruct((NUM_MOL, 1), jnp.int32)],
        name="tc_pool")(v0, v1, b1s, v2m)

    return (molf, molf, moli)
